# baseline (speedup 1.0000x reference)
import functools

import jax
import jax.numpy as jnp
from jax import lax
from jax.experimental import pallas as pl
from jax.experimental.pallas import tpu as pltpu
from jax.experimental.pallas import tpu_sc as plsc

_D = 1024
_V = 11
_NW = 32
_CH = 16
_LANES = 16
_NBUF = 4
_NEB = 4


def _make_sc_kernel(n_rows):
    rows_w = n_rows // _NW
    nch = rows_w // _CH
    mesh = plsc.VectorSubcoreMesh(core_axis_name="c", subcore_axis_name="s")
    xbuf = pltpu.VMEM((_CH, _D), jnp.float32)
    ebuf = pltpu.VMEM((_CH, _D // 2), jnp.int32)
    sem = pltpu.SemaphoreType.DMA

    @functools.partial(
        pl.kernel,
        mesh=mesh,
        out_type=jax.ShapeDtypeStruct((n_rows, _D), jnp.float32),
        scratch_types=(
            [pltpu.VMEM((rows_w,), jnp.int32)]
            + [xbuf] * _NBUF + [ebuf] * _NEB
            + [sem] * _NBUF + [sem] * _NEB + [sem] * _NBUF
        ),
    )
    def k(x_hbm, s_hbm, t_hbm, out_hbm, idx_v, *bufsem):
        xbs = bufsem[:_NBUF]
        ebs = bufsem[_NBUF:_NBUF + _NEB]
        lss = bufsem[_NBUF + _NEB:2 * _NBUF + _NEB]
        gss = bufsem[2 * _NBUF + _NEB:2 * _NBUF + 2 * _NEB]
        sss = bufsem[2 * _NBUF + 2 * _NEB:]
        wid = lax.axis_index("s") * 2 + lax.axis_index("c")
        base = wid * rows_w
        pltpu.sync_copy(s_hbm.at[pl.ds(base, rows_w)], idx_v)

        off = wid * _V

        def shift(j, cc):
            sl = pl.ds(j * _LANES, _LANES)
            idx_v[sl] = idx_v[sl] + off
            return cc

        lax.fori_loop(0, rows_w // _LANES, shift, 0, unroll=8)

        def issue_load(i, b):
            pltpu.async_copy(x_hbm.at[pl.ds(base + i * _CH, _CH)],
                             xbs[b], lss[b])

        def issue_gath(i, e):
            pltpu.async_copy(t_hbm.at[idx_v.at[pl.ds(i * _CH, _CH)]],
                             ebs[e], gss[e])

        for b in range(_NBUF):
            issue_load(b, b)
        for e in range(_NEB):
            issue_gath(e, e)

        def quad(i4, carry):
            for b in range(_NBUF):
                i = i4 * _NBUF + b
                e = b % _NEB
                pltpu.make_async_copy(x_hbm.at[pl.ds(0, _CH)],
                                      xbs[b], lss[b]).wait()
                pltpu.make_async_copy(x_hbm.at[pl.ds(0, _CH)],
                                      ebs[e], gss[e]).wait()
                xb, eb = xbs[b], ebs[e]

                def row(r, rc):
                    def col(c, cc):
                        ev = eb[r, pl.ds(c * _LANES, _LANES)]
                        lo = lax.bitcast_convert_type(ev << 16, jnp.float32)
                        hi = lax.bitcast_convert_type(ev & jnp.int32(-65536),
                                                      jnp.float32)
                        plsc.addupdate(
                            xb.at[r, pl.ds(c * 2 * _LANES, _LANES)], lo)
                        plsc.addupdate(
                            xb.at[r, pl.ds(c * 2 * _LANES + _LANES, _LANES)],
                            hi)
                        return cc
                    lax.fori_loop(0, _D // (2 * _LANES), col, 0, unroll=8)
                    return rc

                lax.fori_loop(0, _CH, row, 0)

                @pl.when(i + _NEB < nch)
                def _():
                    issue_gath(i + _NEB, e)
                pltpu.async_copy(xb, out_hbm.at[pl.ds(base + i * _CH, _CH)],
                                 sss[b])
                jb = (b + _NBUF - 1) % _NBUF

                @pl.when(jnp.logical_and(i >= 1, i + _NBUF - 1 < nch))
                def _():
                    pltpu.make_async_copy(xbs[jb], out_hbm.at[pl.ds(0, _CH)],
                                          sss[jb]).wait()
                    issue_load(i + _NBUF - 1, jb)
            return carry

        lax.fori_loop(0, nch // _NBUF, quad, 0)
        for b in range(_NBUF):
            pltpu.make_async_copy(xbs[b], out_hbm.at[pl.ds(0, _CH)],
                                  sss[b]).wait()

    return k


def kernel(x, scores, score_embeddings):
    B, L, D = x.shape
    n = B * L
    xf = x.reshape(n, D)
    sf = scores.reshape(n).astype(jnp.int32)
    tab_bf = score_embeddings.astype(jnp.bfloat16)
    t2 = (tab_bf.reshape(_V, _D // (2 * _LANES), 2, _LANES)
          .swapaxes(2, 3).reshape(_V, _D))
    t3 = lax.bitcast_convert_type(t2.reshape(_V, _D // 2, 2), jnp.int32)
    t_rep = jnp.tile(t3, (_NW, 1))
    out = _make_sc_kernel(n)(xf, sf, t_rep)
    return out.reshape(B, L, D)


# speedup vs baseline: 1.0053x; 1.0053x over previous
import functools

import jax
import jax.numpy as jnp
from jax import lax
from jax.experimental import pallas as pl
from jax.experimental.pallas import tpu as pltpu
from jax.experimental.pallas import tpu_sc as plsc

_D = 1024
_V = 11
_NW = 32
_CH = 16
_LANES = 16
_NBUF = 4
_NEB = 2


def _make_sc_kernel(n_rows):
    rows_w = n_rows // _NW
    nch = rows_w // _CH
    mesh = plsc.VectorSubcoreMesh(core_axis_name="c", subcore_axis_name="s")
    xbuf = pltpu.VMEM((_CH, _D), jnp.float32)
    ebuf = pltpu.VMEM((_CH, _D // 2), jnp.int32)
    sem = pltpu.SemaphoreType.DMA

    @functools.partial(
        pl.kernel,
        mesh=mesh,
        out_type=jax.ShapeDtypeStruct((n_rows, _D), jnp.float32),
        scratch_types=(
            [pltpu.VMEM((rows_w,), jnp.int32)]
            + [xbuf] * _NBUF + [ebuf] * _NEB
            + [sem] * _NBUF + [sem] * _NEB + [sem] * _NBUF
        ),
    )
    def k(x_hbm, s_hbm, t_hbm, out_hbm, idx_v, *bufsem):
        xbs = bufsem[:_NBUF]
        ebs = bufsem[_NBUF:_NBUF + _NEB]
        lss = bufsem[_NBUF + _NEB:2 * _NBUF + _NEB]
        gss = bufsem[2 * _NBUF + _NEB:2 * _NBUF + 2 * _NEB]
        sss = bufsem[2 * _NBUF + 2 * _NEB:]
        wid = lax.axis_index("s") * 2 + lax.axis_index("c")
        base = wid * rows_w
        pltpu.sync_copy(s_hbm.at[pl.ds(base, rows_w)], idx_v)

        off = wid * _V

        def shift(j, cc):
            sl = pl.ds(j * _LANES, _LANES)
            idx_v[sl] = idx_v[sl] + off
            return cc

        lax.fori_loop(0, rows_w // _LANES, shift, 0, unroll=8)

        def issue_load(i, b):
            pltpu.async_copy(x_hbm.at[pl.ds(base + i * _CH, _CH)],
                             xbs[b], lss[b])

        def issue_gath(i, e):
            pltpu.async_copy(t_hbm.at[idx_v.at[pl.ds(i * _CH, _CH)]],
                             ebs[e], gss[e])

        for b in range(_NBUF):
            issue_load(b, b)
        for e in range(_NEB):
            issue_gath(e, e)

        def quad(i4, carry):
            for b in range(_NBUF):
                i = i4 * _NBUF + b
                e = b % _NEB
                pltpu.make_async_copy(x_hbm.at[pl.ds(0, _CH)],
                                      xbs[b], lss[b]).wait()
                pltpu.make_async_copy(x_hbm.at[pl.ds(0, _CH)],
                                      ebs[e], gss[e]).wait()
                xb, eb = xbs[b], ebs[e]

                def row(r, rc):
                    def col(c, cc):
                        ev = eb[r, pl.ds(c * _LANES, _LANES)]
                        lo = lax.bitcast_convert_type(ev << 16, jnp.float32)
                        hi = lax.bitcast_convert_type(ev & jnp.int32(-65536),
                                                      jnp.float32)
                        plsc.addupdate(
                            xb.at[r, pl.ds(c * 2 * _LANES, _LANES)], lo)
                        plsc.addupdate(
                            xb.at[r, pl.ds(c * 2 * _LANES + _LANES, _LANES)],
                            hi)
                        return cc
                    lax.fori_loop(0, _D // (2 * _LANES), col, 0, unroll=8)
                    return rc

                lax.fori_loop(0, _CH, row, 0)

                @pl.when(i + _NEB < nch)
                def _():
                    issue_gath(i + _NEB, e)
                pltpu.async_copy(xb, out_hbm.at[pl.ds(base + i * _CH, _CH)],
                                 sss[b])
                jb = (b + _NBUF - 1) % _NBUF

                @pl.when(jnp.logical_and(i >= 1, i + _NBUF - 1 < nch))
                def _():
                    pltpu.make_async_copy(xbs[jb], out_hbm.at[pl.ds(0, _CH)],
                                          sss[jb]).wait()
                    issue_load(i + _NBUF - 1, jb)
            return carry

        lax.fori_loop(0, nch // _NBUF, quad, 0)
        for b in range(_NBUF):
            pltpu.make_async_copy(xbs[b], out_hbm.at[pl.ds(0, _CH)],
                                  sss[b]).wait()

    return k


def kernel(x, scores, score_embeddings):
    B, L, D = x.shape
    n = B * L
    xf = x.reshape(n, D)
    sf = scores.reshape(n).astype(jnp.int32)
    tab_bf = score_embeddings.astype(jnp.bfloat16)
    t2 = (tab_bf.reshape(_V, _D // (2 * _LANES), 2, _LANES)
          .swapaxes(2, 3).reshape(_V, _D))
    t3 = lax.bitcast_convert_type(t2.reshape(_V, _D // 2, 2), jnp.int32)
    t_rep = jnp.tile(t3, (_NW, 1))
    out = _make_sc_kernel(n)(xf, sf, t_rep)
    return out.reshape(B, L, D)
